# baseline (speedup 1.0000x reference)
import functools

import jax
import jax.numpy as jnp
from jax import lax
from jax.experimental import pallas as pl
from jax.experimental.pallas import tpu as pltpu
from jax.experimental.pallas import tpu_sc as plsc

B, N, D = 1024, 128, 128
RC = 64
MC = 32

NC, NS = 2, 16
NW = NC * NS
BPW = B // NW
ZWORDS = N * D
SLAB = BPW * ZWORDS


def _rank_body(rand_ref, off_ref):
    i = pl.program_id(0)
    r = rand_ref[...]
    rj = r[:, None, :]
    rk = r[:, :, None]
    k_iota = lax.broadcasted_iota(jnp.int32, (RC, N, N), 1)
    j_iota = lax.broadcasted_iota(jnp.int32, (RC, N, N), 2)
    lt = (rk < rj) | ((rk == rj) & (k_iota < j_iota))
    ranks = jnp.sum(lt.astype(jnp.int32), axis=1)
    bidx = i * RC + lax.broadcasted_iota(jnp.int32, (RC, N), 0)
    jj = lax.broadcasted_iota(jnp.int32, (RC, N), 1)
    off_ref[...] = bidx * (N * D) + ranks * D + jj


def _row_body(row_ref):
    row_ref[...] = jnp.zeros((MC, N, D), jnp.float32)


def _sc_col_body(off_hbm, out_hbm, zbuf, idx_v, ones_v, zsem, ssem):
    wid = lax.axis_index("s") * NC + lax.axis_index("c")
    base = wid * SLAB

    def zstep(i, carry):
        zbuf[pl.ds(i * 16, 16)] = jnp.zeros((16,), jnp.float32)
        return carry

    lax.fori_loop(0, ZWORDS // 16, zstep, 0)
    for c in range(N // 16):
        ones_v[pl.ds(c * 16, 16)] = jnp.ones((16,), jnp.float32)

    zero_copies = [
        pltpu.async_copy(zbuf, out_hbm.at[pl.ds(base + t * ZWORDS, ZWORDS)], zsem)
        for t in range(BPW)
    ]
    pltpu.sync_copy(off_hbm.at[wid], idx_v)
    for cp in zero_copies:
        cp.wait()
    scatter_copies = [
        pltpu.async_copy(ones_v, out_hbm.at[idx_v.at[t]], ssem) for t in range(BPW)
    ]
    for cp in scatter_copies:
        cp.wait()


_sc_col = functools.partial(
    pl.kernel,
    out_type=jax.ShapeDtypeStruct((B * N * D,), jnp.float32),
    mesh=plsc.VectorSubcoreMesh(core_axis_name="c", subcore_axis_name="s"),
    scratch_types=[
        pltpu.VMEM((ZWORDS,), jnp.float32),
        pltpu.VMEM((BPW, N), jnp.int32),
        pltpu.VMEM((N,), jnp.float32),
        pltpu.SemaphoreType.DMA,
        pltpu.SemaphoreType.DMA,
    ],
)(_sc_col_body)


def kernel(distance_matrix):
    rand = jax.random.uniform(jax.random.key(42), (B, N), dtype=jnp.float32)
    off = pl.pallas_call(
        _rank_body,
        grid=(B // RC,),
        in_specs=[pl.BlockSpec((RC, N), lambda i: (i, 0))],
        out_specs=pl.BlockSpec((RC, N), lambda i: (i, 0)),
        out_shape=jax.ShapeDtypeStruct((B, N), jnp.int32),
    )(rand)
    col_flat = _sc_col(off.reshape(NW, BPW, N))
    row_emb = pl.pallas_call(
        _row_body,
        grid=(B // MC,),
        out_specs=pl.BlockSpec((MC, N, D), lambda i: (i, 0, 0)),
        out_shape=jax.ShapeDtypeStruct((B, N, D), jnp.float32),
    )()
    return (row_emb, col_flat.reshape(B, N, D), distance_matrix)
